# baseline (speedup 1.0000x reference)
import functools
from math import sqrt

import jax
import jax.numpy as jnp
from jax import lax
from jax.experimental import pallas as pl
from jax.experimental.pallas import tpu as pltpu
from jax.experimental.pallas import tpu_sc as plsc

_MODEL_DIM = 64
_SCALE = sqrt(_MODEL_DIM)


def _make_sc_gather(vocab, dim, n_seq, n_batch):
    info = plsc.get_sparse_core_info()
    nc, ns, lanes = info.num_cores, info.num_subcores, info.num_lanes
    nw = nc * ns
    assert n_batch % nw == 0
    slab = n_batch // nw
    ch = slab // 2
    n_chunks = 2 * n_seq
    n_tok = n_seq * n_batch
    mesh = plsc.VectorSubcoreMesh(core_axis_name="c", subcore_axis_name="s")

    @functools.partial(
        pl.kernel,
        mesh=mesh,
        compiler_params=pltpu.CompilerParams(
            use_tc_tiling_on_sc=True, needs_layout_passes=False
        ),
        out_type=(
            jax.ShapeDtypeStruct((n_tok, 2 * dim), jnp.float32),
            jax.ShapeDtypeStruct((n_tok,), jnp.int32),
        ),
        scratch_types=[
            pltpu.VMEM((ch,), jnp.int32),
            pltpu.VMEM((ch,), jnp.int32),
            pltpu.VMEM((ch,), jnp.int32),
            pltpu.VMEM((ch,), jnp.int32),
            pltpu.VMEM((ch, 2 * dim), jnp.float32),
            pltpu.VMEM((ch, 2 * dim), jnp.float32),
            pltpu.SemaphoreType.DMA,
            pltpu.SemaphoreType.DMA,
            pltpu.SemaphoreType.DMA,
            pltpu.SemaphoreType.DMA,
        ],
    )
    def k(idx_hbm, table_hbm, m_hbm, par_hbm,
          i0, i1, p0, p1, a0, a1, g0, g1, o0, o1):
        wid = lax.axis_index("s") * nc + lax.axis_index("c")
        base = wid * slab
        ibuf, pbuf, abuf = (i0, i1), (p0, p1), (a0, a1)
        gs, os = (g0, g1), (o0, o1)

        def tok0(c):
            return (c // 2) * n_batch + base + (c % 2) * ch

        def idx_stage(c, b):
            pltpu.sync_copy(
                idx_hbm.at[c // 2, pl.ds(base + (c % 2) * ch, ch)], ibuf[b]
            )

            def prow(j, cc):
                sl = pl.ds(j * lanes, lanes)
                v16 = ibuf[b][sl]
                pbuf[b][sl] = lax.shift_right_logical(v16, 1)
                ibuf[b][sl] = v16 & 1
                return cc

            lax.fori_loop(0, ch // lanes, prow, 0, unroll=4)

        def gather(b):
            pltpu.async_copy(table_hbm.at[pbuf[b]], abuf[b], gs[b])

        def gwait(b):
            pltpu.make_async_copy(table_hbm.at[pbuf[b]], abuf[b], gs[b]).wait()

        def out_start(c, b):
            pltpu.async_copy(abuf[b], m_hbm.at[pl.ds(tok0(c), ch)], os[b])
            pltpu.async_copy(ibuf[b], par_hbm.at[pl.ds(tok0(c), ch)], os[b])

        def out_wait(b):
            pltpu.make_async_copy(
                abuf[b], m_hbm.at[pl.ds(base, ch)], os[b]
            ).wait()
            pltpu.make_async_copy(
                ibuf[b], par_hbm.at[pl.ds(base, ch)], os[b]
            ).wait()

        idx_stage(0, 0)
        gather(0)

        def body(g, carry):
            c0 = 2 * g
            c1 = c0 + 1
            idx_stage(c1, 1)
            gather(1)
            gwait(0)
            out_start(c0, 0)

            @pl.when(c1 + 1 < n_chunks)
            def _():
                out_wait(0)
                idx_stage(c1 + 1, 0)
                gather(0)

            gwait(1)
            out_start(c1, 1)

            @pl.when(c1 + 1 < n_chunks)
            def _():
                out_wait(1)

            return carry

        lax.fori_loop(0, n_chunks // 2, body, 0)
        out_wait(0)
        out_wait(1)

    return k


def _tc_select_transpose(m, par, n_seq, dim, n_batch, blk):
    n_blk = n_batch // blk

    def body(m_ref, par_ref, out_ref):
        rows = m_ref[...]
        left = rows[:, :dim].T
        right = rows[:, dim:].T
        sel = par_ref[0] != 0
        vals = jnp.where(sel, right, left) * _SCALE
        out_ref[...] = vals[None]

    return pl.pallas_call(
        body,
        grid=(n_seq, n_blk),
        in_specs=[
            pl.BlockSpec((blk, 2 * dim), lambda t, j: (t * n_blk + j, 0)),
            pl.BlockSpec((1, 1, blk), lambda t, j: (t * n_blk + j, 0, 0)),
        ],
        out_specs=pl.BlockSpec((1, dim, blk), lambda t, j: (t, 0, j)),
        out_shape=jax.ShapeDtypeStruct((n_seq, dim, n_batch), jnp.float32),
    )(m, par.reshape(n_seq * n_blk, 1, blk))


def kernel(inputs, weight):
    b, t = inputs.shape
    vocab, dim = weight.shape
    idx_t = inputs.T.astype(jnp.int32)
    table2 = weight.reshape(vocab // 2, 2 * dim)
    gather = _make_sc_gather(vocab, dim, t, b)
    m, par = gather(idx_t, table2)
    out_t = _tc_select_transpose(m, par, t, dim, b, 512)
    return out_t.transpose(2, 0, 1)


# speedup vs baseline: 1.3116x; 1.3116x over previous
import functools
from math import sqrt

import jax
import jax.numpy as jnp
from jax import lax
from jax.experimental import pallas as pl
from jax.experimental.pallas import tpu as pltpu
from jax.experimental.pallas import tpu_sc as plsc

_MODEL_DIM = 64
_SCALE = sqrt(_MODEL_DIM)


def _make_sc_gather(vocab, dim, n_seq, n_batch):
    info = plsc.get_sparse_core_info()
    nc, ns, lanes = info.num_cores, info.num_subcores, info.num_lanes
    nw = nc * ns
    assert n_batch % nw == 0
    slab = n_batch // nw
    ch = slab // 2
    n_chunks = 2 * n_seq
    n_tok = n_seq * n_batch
    mesh = plsc.VectorSubcoreMesh(core_axis_name="c", subcore_axis_name="s")

    @functools.partial(
        pl.kernel,
        mesh=mesh,
        compiler_params=pltpu.CompilerParams(
            use_tc_tiling_on_sc=True, needs_layout_passes=False
        ),
        out_type=(
            jax.ShapeDtypeStruct((n_tok, 2 * dim), jnp.float32),
            jax.ShapeDtypeStruct((n_tok,), jnp.int32),
        ),
        scratch_types=[
            pltpu.VMEM((ch,), jnp.int32),
            pltpu.VMEM((ch,), jnp.int32),
            pltpu.VMEM((ch,), jnp.int32),
            pltpu.VMEM((ch,), jnp.int32),
            pltpu.VMEM((ch, 2 * dim), jnp.float32),
            pltpu.VMEM((ch, 2 * dim), jnp.float32),
            pltpu.SemaphoreType.DMA,
            pltpu.SemaphoreType.DMA,
            pltpu.SemaphoreType.DMA,
            pltpu.SemaphoreType.DMA,
        ],
    )
    def k(idx_hbm, table_hbm, m_hbm, par_hbm,
          i0, i1, p0, p1, a0, a1, g0, g1, o0, o1):
        wid = lax.axis_index("s") * nc + lax.axis_index("c")
        base = wid * slab
        ibuf, pbuf, abuf = (i0, i1), (p0, p1), (a0, a1)
        gs, os = (g0, g1), (o0, o1)

        def tok0(c):
            return (c // 2) * n_batch + base + (c % 2) * ch

        def idx_stage(c, b):
            pltpu.sync_copy(
                idx_hbm.at[c // 2, pl.ds(base + (c % 2) * ch, ch)], ibuf[b]
            )

            def prow(j, cc):
                sl = pl.ds(j * lanes, lanes)
                v16 = ibuf[b][sl]
                pbuf[b][sl] = lax.shift_right_logical(v16, 1)
                ibuf[b][sl] = v16 & 1
                return cc

            lax.fori_loop(0, ch // lanes, prow, 0, unroll=4)

        def gather(b):
            pltpu.async_copy(table_hbm.at[pbuf[b]], abuf[b], gs[b])

        def gwait(b):
            pltpu.make_async_copy(table_hbm.at[pbuf[b]], abuf[b], gs[b]).wait()

        def out_start(c, b):
            pltpu.async_copy(abuf[b], m_hbm.at[pl.ds(tok0(c), ch)], os[b])
            pltpu.async_copy(ibuf[b], par_hbm.at[pl.ds(tok0(c), ch)], os[b])

        def out_wait(b):
            pltpu.make_async_copy(
                abuf[b], m_hbm.at[pl.ds(base, ch)], os[b]
            ).wait()
            pltpu.make_async_copy(
                ibuf[b], par_hbm.at[pl.ds(base, ch)], os[b]
            ).wait()

        idx_stage(0, 0)
        gather(0)

        def body(g, carry):
            c0 = 2 * g
            c1 = c0 + 1
            idx_stage(c1, 1)
            gather(1)
            gwait(0)
            out_start(c0, 0)

            @pl.when(c1 + 1 < n_chunks)
            def _():
                out_wait(0)
                idx_stage(c1 + 1, 0)
                gather(0)

            gwait(1)
            out_start(c1, 1)

            @pl.when(c1 + 1 < n_chunks)
            def _():
                out_wait(1)

            return carry

        lax.fori_loop(0, n_chunks // 2, body, 0)
        out_wait(0)
        out_wait(1)

    return k


def _tc_select_transpose(m, par, n_seq, dim, n_batch, blk):
    n_blk = n_batch // blk

    eye = jnp.eye(2 * dim, dtype=jnp.float32)

    def body(eye_ref, m_ref, par_ref, out_ref):
        rows = m_ref[...]
        rows_t = lax.dot_general(
            eye_ref[...], rows,
            (((1,), (1,)), ((), ())),
            preferred_element_type=jnp.float32,
        )
        sel = par_ref[0] != 0
        vals = jnp.where(sel, rows_t[dim:], rows_t[:dim]) * _SCALE
        out_ref[...] = vals[None]

    return pl.pallas_call(
        body,
        grid=(n_seq, n_blk),
        in_specs=[
            pl.BlockSpec((2 * dim, 2 * dim), lambda t, j: (0, 0)),
            pl.BlockSpec((blk, 2 * dim), lambda t, j: (t * n_blk + j, 0)),
            pl.BlockSpec((1, 1, blk), lambda t, j: (t * n_blk + j, 0, 0)),
        ],
        out_specs=pl.BlockSpec((1, dim, blk), lambda t, j: (t, 0, j)),
        out_shape=jax.ShapeDtypeStruct((n_seq, dim, n_batch), jnp.float32),
        compiler_params=pltpu.CompilerParams(
            dimension_semantics=("parallel", "parallel"),
        ),
    )(eye, m, par.reshape(n_seq * n_blk, 1, blk))


def kernel(inputs, weight):
    b, t = inputs.shape
    vocab, dim = weight.shape
    idx_t = inputs.T.astype(jnp.int32)
    table2 = weight.reshape(vocab // 2, 2 * dim)
    gather = _make_sc_gather(vocab, dim, t, b)
    m, par = gather(idx_t, table2)
    out_t = _tc_select_transpose(m, par, t, dim, b, 1024)
    return out_t.transpose(2, 0, 1)


# speedup vs baseline: 1.3129x; 1.0010x over previous
import functools
from math import sqrt

import jax
import jax.numpy as jnp
from jax import lax
from jax.experimental import pallas as pl
from jax.experimental.pallas import tpu as pltpu
from jax.experimental.pallas import tpu_sc as plsc

_MODEL_DIM = 64
_SCALE = sqrt(_MODEL_DIM)


def _make_sc_gather(vocab, dim, n_seq, n_batch):
    info = plsc.get_sparse_core_info()
    nc, ns, lanes = info.num_cores, info.num_subcores, info.num_lanes
    nw = nc * ns
    assert n_batch % nw == 0
    slab = n_batch // nw
    ch = slab // 2
    n_chunks = 2 * n_seq
    n_tok = n_seq * n_batch
    mesh = plsc.VectorSubcoreMesh(core_axis_name="c", subcore_axis_name="s")

    @functools.partial(
        pl.kernel,
        mesh=mesh,
        compiler_params=pltpu.CompilerParams(
            use_tc_tiling_on_sc=True, needs_layout_passes=False
        ),
        out_type=(
            jax.ShapeDtypeStruct((n_tok, 2 * dim), jnp.float32),
            jax.ShapeDtypeStruct((n_tok,), jnp.int32),
        ),
        scratch_types=[
            pltpu.VMEM((ch,), jnp.int32),
            pltpu.VMEM((ch,), jnp.int32),
            pltpu.VMEM((ch,), jnp.int32),
            pltpu.VMEM((ch,), jnp.int32),
            pltpu.VMEM((ch, 2 * dim), jnp.float32),
            pltpu.VMEM((ch, 2 * dim), jnp.float32),
            pltpu.SemaphoreType.DMA,
            pltpu.SemaphoreType.DMA,
            pltpu.SemaphoreType.DMA,
            pltpu.SemaphoreType.DMA,
        ],
    )
    def k(idx_hbm, table_hbm, m_hbm, par_hbm,
          i0, i1, p0, p1, a0, a1, g0, g1, o0, o1):
        wid = lax.axis_index("s") * nc + lax.axis_index("c")
        base = wid * slab
        ibuf, pbuf, abuf = (i0, i1), (p0, p1), (a0, a1)
        gs, os = (g0, g1), (o0, o1)

        def tok0(c):
            return (c // 2) * n_batch + base + (c % 2) * ch

        def idx_stage(c, b):
            pltpu.sync_copy(
                idx_hbm.at[c // 2, pl.ds(base + (c % 2) * ch, ch)], ibuf[b]
            )

            def prow(j, cc):
                sl = pl.ds(j * lanes, lanes)
                v16 = ibuf[b][sl]
                pbuf[b][sl] = lax.shift_right_logical(v16, 1)
                ibuf[b][sl] = v16 & 1
                return cc

            lax.fori_loop(0, ch // lanes, prow, 0, unroll=4)

        def gather(b):
            pltpu.async_copy(table_hbm.at[pbuf[b]], abuf[b], gs[b])

        def gwait(b):
            pltpu.make_async_copy(table_hbm.at[pbuf[b]], abuf[b], gs[b]).wait()

        def out_start(c, b):
            pltpu.async_copy(abuf[b], m_hbm.at[pl.ds(tok0(c), ch)], os[b])
            pltpu.async_copy(ibuf[b], par_hbm.at[pl.ds(tok0(c), ch)], os[b])

        def out_wait(b):
            pltpu.make_async_copy(
                abuf[b], m_hbm.at[pl.ds(base, ch)], os[b]
            ).wait()
            pltpu.make_async_copy(
                ibuf[b], par_hbm.at[pl.ds(base, ch)], os[b]
            ).wait()

        idx_stage(0, 0)
        gather(0)

        def body(g, carry):
            c0 = 2 * g
            c1 = c0 + 1
            idx_stage(c1, 1)
            gather(1)
            gwait(0)
            out_start(c0, 0)

            @pl.when(c1 + 1 < n_chunks)
            def _():
                out_wait(0)
                idx_stage(c1 + 1, 0)
                gather(0)

            gwait(1)
            out_start(c1, 1)

            @pl.when(c1 + 1 < n_chunks)
            def _():
                out_wait(1)

            return carry

        lax.fori_loop(0, n_chunks // 2, body, 0)
        out_wait(0)
        out_wait(1)

    return k


def _tc_select_transpose(m, par, n_seq, dim, n_batch, blk):
    n_blk = n_batch // blk

    eye = jnp.eye(2 * dim, dtype=jnp.bfloat16)

    def body(eye_ref, m_ref, par_ref, out_ref):
        rows = m_ref[...]
        rows_t = lax.dot_general(
            eye_ref[...], rows.astype(jnp.bfloat16),
            (((1,), (1,)), ((), ())),
            preferred_element_type=jnp.float32,
        )
        sel = par_ref[0] != 0
        vals = jnp.where(sel, rows_t[dim:], rows_t[:dim]) * _SCALE
        out_ref[...] = vals[None]

    return pl.pallas_call(
        body,
        grid=(n_seq, n_blk),
        in_specs=[
            pl.BlockSpec((2 * dim, 2 * dim), lambda t, j: (0, 0)),
            pl.BlockSpec((blk, 2 * dim), lambda t, j: (t * n_blk + j, 0)),
            pl.BlockSpec((1, 1, blk), lambda t, j: (t * n_blk + j, 0, 0)),
        ],
        out_specs=pl.BlockSpec((1, dim, blk), lambda t, j: (t, 0, j)),
        out_shape=jax.ShapeDtypeStruct((n_seq, dim, n_batch), jnp.float32),
        compiler_params=pltpu.CompilerParams(
            dimension_semantics=("parallel", "parallel"),
        ),
    )(eye, m, par.reshape(n_seq * n_blk, 1, blk))


def kernel(inputs, weight):
    b, t = inputs.shape
    vocab, dim = weight.shape
    idx_t = inputs.T.astype(jnp.int32)
    table2 = weight.reshape(vocab // 2, 2 * dim)
    gather = _make_sc_gather(vocab, dim, t, b)
    m, par = gather(idx_t, table2)
    out_t = _tc_select_transpose(m, par, t, dim, b, 1024)
    return out_t.transpose(2, 0, 1)


# speedup vs baseline: 1.5660x; 1.1927x over previous
import functools
from math import sqrt

import jax
import jax.numpy as jnp
from jax import lax
from jax.experimental import pallas as pl
from jax.experimental.pallas import tpu as pltpu
from jax.experimental.pallas import tpu_sc as plsc

_MODEL_DIM = 64
_SCALE = sqrt(_MODEL_DIM)


def _make_sc_lookup(vocab, dim, n_tokens):
    info = plsc.get_sparse_core_info()
    nc, ns, lanes = info.num_cores, info.num_subcores, info.num_lanes
    nw = nc * ns
    assert n_tokens % nw == 0
    per_w = n_tokens // nw
    chunk = 512
    while per_w % (2 * chunk):
        chunk //= 2
    n_chunks = per_w // chunk
    mesh = plsc.VectorSubcoreMesh(core_axis_name="c", subcore_axis_name="s")

    @functools.partial(
        pl.kernel,
        mesh=mesh,
        compiler_params=pltpu.CompilerParams(use_tc_tiling_on_sc=False),
        out_type=jax.ShapeDtypeStruct((n_tokens, dim), jnp.float32),
        scratch_types=[
            pltpu.VMEM((n_chunks, chunk), jnp.int32),
            pltpu.VMEM((chunk, dim), jnp.float32),
            pltpu.VMEM((chunk, dim), jnp.float32),
            pltpu.SemaphoreType.DMA,
            pltpu.SemaphoreType.DMA,
            pltpu.SemaphoreType.DMA,
            pltpu.SemaphoreType.DMA,
        ],
    )
    def k(idx_hbm, table_hbm, out_hbm, idx_v, rows0, rows1, g0, g1, o0, o1):
        wid = lax.axis_index("s") * nc + lax.axis_index("c")
        base = wid * per_w
        rows = (rows0, rows1)
        gsem = (g0, g1)
        osem = (o0, o1)

        pltpu.sync_copy(
            idx_hbm.at[pl.ds(wid * n_chunks, n_chunks)], idx_v
        )

        def gather(i, b):
            pltpu.async_copy(table_hbm.at[idx_v.at[i]], rows[b], gsem[b])

        def gather_wait(b):
            pltpu.make_async_copy(table_hbm.at[idx_v.at[0]], rows[b], gsem[b]).wait()

        def out_start(i, b):
            pltpu.async_copy(rows[b], out_hbm.at[pl.ds(base + i * chunk, chunk)], osem[b])

        def out_wait(b):
            pltpu.make_async_copy(
                rows[b], out_hbm.at[pl.ds(base, chunk)], osem[b]
            ).wait()

        def scale(b):
            r = rows[b]

            def srow(row, c2):
                for q in range(dim // lanes):
                    sl = pl.ds(q * lanes, lanes)
                    r[row, sl] = r[row, sl] * _SCALE
                return c2

            lax.fori_loop(0, chunk, srow, 0, unroll=4)

        gather(0, 0)

        def body(g, carry):
            i0 = g * 2
            @pl.when(i0 > 0)
            def _():
                out_wait(1)

            gather(i0 + 1, 1)
            gather_wait(0)
            scale(0)
            out_start(i0, 0)
            out_wait(0)

            @pl.when(i0 + 2 < n_chunks)
            def _():
                gather(i0 + 2, 0)

            gather_wait(1)
            scale(1)
            out_start(i0 + 1, 1)
            return carry

        lax.fori_loop(0, n_chunks // 2, body, 0)
        out_wait(1)

    return k


def kernel(inputs, weight):
    b, t = inputs.shape
    vocab, dim = weight.shape
    n_tokens = b * t
    lookup = _make_sc_lookup(vocab, dim, n_tokens)
    info = plsc.get_sparse_core_info()
    nw = info.num_cores * info.num_subcores
    per_w = n_tokens // nw
    chunk = 512
    while per_w % (2 * chunk):
        chunk //= 2
    idx = inputs.reshape(n_tokens // chunk, chunk).astype(jnp.int32)
    out = lookup(idx, weight)
    return out.reshape(b, t, dim)
